# baseline (device time: 6051 ns/iter reference)
import jax
import jax.numpy as jnp
from jax import lax
from jax.experimental import pallas as pl
from jax.experimental.pallas import tpu as pltpu

N_DEV = 4


def kernel(x):
    m_per, n = x.shape

    def body(x_ref, out_ref, xv_ref, res_ref, gather_ref, send_sems,
             recv_sems, in_sem, out_sem):
        my = lax.axis_index("i")

        in_dma = pltpu.make_async_copy(x_ref, xv_ref, in_sem)
        in_dma.start()

        def send_rdma(i, j):
            return pltpu.make_async_remote_copy(
                src_ref=gather_ref.at[i],
                dst_ref=gather_ref.at[i],
                send_sem=send_sems.at[j],
                recv_sem=recv_sems.at[i],
                device_id=(j,),
                device_id_type=pl.DeviceIdType.MESH,
            )

        barrier_sem = pltpu.get_barrier_semaphore()
        for j in range(1, N_DEV):
            @pl.when(my == j)
            def _():
                for i in range(j):
                    pl.semaphore_signal(
                        barrier_sem, inc=1,
                        device_id=(i,), device_id_type=pl.DeviceIdType.MESH,
                    )

        in_dma.wait()
        total = jnp.sum(xv_ref[:, :], axis=0, keepdims=True)
        for i in range(N_DEV):
            @pl.when(my == i)
            def _():
                gather_ref[i, :, :] = total

        for i in range(N_DEV - 1):
            @pl.when(my == i)
            def _():
                pl.semaphore_wait(barrier_sem, N_DEV - 1 - i)
                for j in range(i + 1, N_DEV):
                    send_rdma(i, j).start()

        row = lax.broadcasted_iota(jnp.int32, (m_per, m_per), 0)
        col = lax.broadcasted_iota(jnp.int32, (m_per, m_per), 1)
        tril = (col <= row).astype(jnp.bfloat16)
        local = jnp.dot(tril, xv_ref[:, :].astype(jnp.bfloat16),
                        preferred_element_type=jnp.float32)

        for j in range(1, N_DEV):
            @pl.when(my == j)
            def _():
                for i in range(j):
                    send_rdma(i, j).wait_recv()

        offset = jnp.zeros((1, n), jnp.float32)
        for i in range(N_DEV - 1):
            offset = offset + jnp.where(my > i, gather_ref[i, :, :], 0.0)

        res_ref[:, :] = (local + offset).astype(jnp.bfloat16)
        out_dma = pltpu.make_async_copy(res_ref, out_ref, out_sem)
        out_dma.start()

        for i in range(N_DEV - 1):
            @pl.when(my == i)
            def _():
                for j in range(i + 1, N_DEV):
                    send_rdma(i, j).wait_send()

        out_dma.wait()

    return pl.pallas_call(
        body,
        out_shape=jax.ShapeDtypeStruct((m_per, n), jnp.bfloat16),
        in_specs=[pl.BlockSpec(memory_space=pl.ANY)],
        out_specs=pl.BlockSpec(memory_space=pl.ANY),
        scratch_shapes=[
            pltpu.VMEM((m_per, n), jnp.float32),
            pltpu.VMEM((m_per, n), jnp.bfloat16),
            pltpu.VMEM((N_DEV, 1, n), jnp.float32),
            pltpu.SemaphoreType.DMA((N_DEV,)),
            pltpu.SemaphoreType.DMA((N_DEV,)),
            pltpu.SemaphoreType.DMA,
            pltpu.SemaphoreType.DMA,
        ],
        compiler_params=pltpu.CompilerParams(collective_id=0),
    )(x)


# device time: 5972 ns/iter; 1.0132x vs baseline; 1.0132x over previous
import jax
import jax.numpy as jnp
from jax import lax
from jax.experimental import pallas as pl
from jax.experimental.pallas import tpu as pltpu

N_DEV = 4


def kernel(x):
    m_per, n = x.shape

    def body(x_ref, out_ref, gather_ref, send_sems, recv_sems):
        my = lax.axis_index("i")

        def send_rdma(i, j):
            return pltpu.make_async_remote_copy(
                src_ref=gather_ref.at[i],
                dst_ref=gather_ref.at[i],
                send_sem=send_sems.at[j],
                recv_sem=recv_sems.at[i],
                device_id=(j,),
                device_id_type=pl.DeviceIdType.MESH,
            )

        barrier_sem = pltpu.get_barrier_semaphore()
        for j in range(1, N_DEV):
            @pl.when(my == j)
            def _():
                for i in range(j):
                    pl.semaphore_signal(
                        barrier_sem, inc=1,
                        device_id=(i,), device_id_type=pl.DeviceIdType.MESH,
                    )

        total = jnp.sum(x_ref[:, :], axis=0, keepdims=True)
        for i in range(N_DEV):
            @pl.when(my == i)
            def _():
                gather_ref[i, :, :] = total

        for i in range(N_DEV - 1):
            @pl.when(my == i)
            def _():
                pl.semaphore_wait(barrier_sem, N_DEV - 1 - i)
                for j in range(i + 1, N_DEV):
                    send_rdma(i, j).start()

        row = lax.broadcasted_iota(jnp.int32, (m_per, m_per), 0)
        col = lax.broadcasted_iota(jnp.int32, (m_per, m_per), 1)
        tril = (col <= row).astype(jnp.bfloat16)
        local = jnp.dot(tril, x_ref[:, :].astype(jnp.bfloat16),
                        preferred_element_type=jnp.float32)

        for j in range(1, N_DEV):
            @pl.when(my == j)
            def _():
                for i in range(j):
                    send_rdma(i, j).wait_recv()

        offset = jnp.zeros((1, n), jnp.float32)
        for i in range(N_DEV - 1):
            offset = offset + jnp.where(my > i, gather_ref[i, :, :], 0.0)

        out_ref[:, :] = (local + offset).astype(jnp.bfloat16)

        for i in range(N_DEV - 1):
            @pl.when(my == i)
            def _():
                for j in range(i + 1, N_DEV):
                    send_rdma(i, j).wait_send()

    return pl.pallas_call(
        body,
        out_shape=jax.ShapeDtypeStruct((m_per, n), jnp.bfloat16),
        in_specs=[pl.BlockSpec(memory_space=pltpu.MemorySpace.VMEM)],
        out_specs=pl.BlockSpec(memory_space=pltpu.MemorySpace.VMEM),
        scratch_shapes=[
            pltpu.VMEM((N_DEV, 1, n), jnp.float32),
            pltpu.SemaphoreType.DMA((N_DEV,)),
            pltpu.SemaphoreType.DMA((N_DEV,)),
        ],
        compiler_params=pltpu.CompilerParams(collective_id=0),
    )(x)
